# baseline (speedup 1.0000x reference)
import jax
import jax.numpy as jnp
from jax import lax
from jax.experimental import pallas as pl
from jax.experimental.pallas import tpu as pltpu

_WIDTHS = (2, 2, 1, 6, 18, 18, 12, 12, 12, 18)
_OUT_SHAPES = ((1, 2), (1, 2), (1, 1), (1, 6), (6, 3), (6, 3),
               (6, 2), (6, 2), (6, 2), (6, 3))


def _body(x_ref, *refs):
    ws = refs[:10]
    outs = refs[10:20]

    v = x_ref[0, 0] * 100.0
    i0 = v.astype(jnp.int32)
    idx = jnp.where(i0.astype(jnp.float32) > v, i0 - 1, i0)

    for (nrows, ncols), d, w, o in zip(_OUT_SHAPES, _WIDTHS, ws, outs):
        val = w[...]
        hit = lax.broadcasted_iota(jnp.int32, (d, 101), 1) == idx
        col = jnp.sum(jnp.where(hit, val, 0.0), axis=1, keepdims=True)
        if nrows == 1:
            o[...] = col
        else:
            for r in range(nrows):
                o[pl.ds(0, ncols), pl.ds(r, 1)] = col[r * ncols:(r + 1) * ncols, :]


_tc_lookup = pl.pallas_call(
    _body,
    out_shape=[jax.ShapeDtypeStruct((c, r), jnp.float32)
               for r, c in _OUT_SHAPES],
    in_specs=[pl.BlockSpec(memory_space=pltpu.SMEM)] +
             [pl.BlockSpec(memory_space=pltpu.VMEM)] * 10,
    out_specs=[pl.BlockSpec(memory_space=pltpu.VMEM)] * 10,
)


def kernel(x, W_enc_embed, W_dec_embed, W_enc_layer, W_dec_layer,
           W_enc_ffn, W_dec_ffn, W_enc_heads, W_dec_heads,
           W_dec_ende_heads, W_dec_arb_ende):
    outs_t = _tc_lookup(x, W_enc_embed.T, W_dec_embed.T, W_enc_layer.T,
                        W_dec_layer.T, W_enc_ffn.T, W_dec_ffn.T,
                        W_enc_heads.T, W_dec_heads.T, W_dec_ende_heads.T,
                        W_dec_arb_ende.T)
    return tuple(o.T for o in outs_t)


# speedup vs baseline: 2.1690x; 2.1690x over previous
import jax
import jax.numpy as jnp
from jax import lax
from jax.experimental import pallas as pl
from jax.experimental.pallas import tpu as pltpu

_WIDTHS = (2, 2, 1, 6, 18, 18, 12, 12, 12, 18)
_OUT_SHAPES = ((1, 2), (1, 2), (1, 1), (1, 6), (6, 3), (6, 3),
               (6, 2), (6, 2), (6, 2), (6, 3))


def _body(x_ref, *refs):
    ws = refs[:10]
    outs = refs[10:20]

    v = x_ref[0, 0] * 100.0
    i0 = v.astype(jnp.int32)
    idx = jnp.where(i0.astype(jnp.float32) > v, i0 - 1, i0)

    for (nrows, ncols), d, w, o in zip(_OUT_SHAPES, _WIDTHS, ws, outs):
        val = w[...]
        hit = lax.broadcasted_iota(jnp.int32, (d, 101), 1) == idx
        col = jnp.sum(jnp.where(hit, val, 0.0), axis=1, keepdims=True)
        if nrows == 1:
            o[...] = col.reshape(1, d)
        else:
            for r in range(nrows):
                o[pl.ds(0, ncols), pl.ds(r, 1)] = col[r * ncols:(r + 1) * ncols, :]


_tc_lookup = pl.pallas_call(
    _body,
    out_shape=[jax.ShapeDtypeStruct((r, c) if r == 1 else (c, r), jnp.float32)
               for r, c in _OUT_SHAPES],
    in_specs=[pl.BlockSpec(memory_space=pltpu.SMEM)] +
             [pl.BlockSpec(memory_space=pltpu.VMEM)] * 10,
    out_specs=[pl.BlockSpec(memory_space=pltpu.VMEM)] * 10,
)


def kernel(x, W_enc_embed, W_dec_embed, W_enc_layer, W_dec_layer,
           W_enc_ffn, W_dec_ffn, W_enc_heads, W_dec_heads,
           W_dec_ende_heads, W_dec_arb_ende):
    outs_t = _tc_lookup(x, W_enc_embed.T, W_dec_embed.T, W_enc_layer.T,
                        W_dec_layer.T, W_enc_ffn.T, W_dec_ffn.T,
                        W_enc_heads.T, W_dec_heads.T, W_dec_ende_heads.T,
                        W_dec_arb_ende.T)
    return tuple(o if s[0] == 1 else o.T
                 for o, s in zip(outs_t, _OUT_SHAPES))
